# Initial kernel scaffold; baseline (speedup 1.0000x reference)
import jax
import jax.numpy as jnp
from jax.experimental import pallas as pl


def kernel(input, support, table):
    raise NotImplementedError("write your pallas kernel here")


# speedup vs baseline: 1.2898x; 1.2898x over previous
import functools

import jax
import jax.numpy as jnp
from jax import lax
from jax.experimental import pallas as pl
from jax.experimental.pallas import tpu as pltpu
from jax.experimental.pallas import tpu_sc as plsc

VOCAB = 1000000
EMBED_DIM = 64
BATCH = 4096
SEQ = 50

_INFO = plsc.get_sparse_core_info()
NC, NS = _INFO.num_cores, _INFO.num_subcores
NW = NC * NS

N_ROWS = BATCH * SEQ
PER_W = N_ROWS // NW
CHUNK = 800
N_CHUNKS = PER_W // CHUNK


def _emb_kernel(in_idx_hbm, sup_idx_hbm, table_hbm, out_in_hbm, out_sup_hbm,
                idx_v, rows_v, sem):
    wid = lax.axis_index("s") * NC + lax.axis_index("c")
    base_w = wid * PER_W

    def body(i, idx_hbm, out_hbm):
        base = base_w + i * CHUNK
        pltpu.sync_copy(idx_hbm.at[pl.ds(base, CHUNK)], idx_v)
        pltpu.async_copy(table_hbm.at[idx_v], rows_v, sem).wait()
        pltpu.sync_copy(rows_v, out_hbm.at[pl.ds(base, CHUNK)])

    lax.fori_loop(0, N_CHUNKS, lambda i, c: (body(i, in_idx_hbm, out_in_hbm), c)[1], 0)
    lax.fori_loop(0, N_CHUNKS, lambda i, c: (body(i, sup_idx_hbm, out_sup_hbm), c)[1], 0)


@jax.jit
def kernel(input, support, table):
    in_flat = input.reshape(N_ROWS).astype(jnp.int32)
    sup_flat = support.reshape(N_ROWS).astype(jnp.int32)

    run = functools.partial(
        pl.kernel,
        out_type=(
            jax.ShapeDtypeStruct((N_ROWS, EMBED_DIM), jnp.float32),
            jax.ShapeDtypeStruct((N_ROWS, EMBED_DIM), jnp.float32),
        ),
        mesh=plsc.VectorSubcoreMesh(core_axis_name="c", subcore_axis_name="s"),
        scratch_types=[
            pltpu.VMEM((CHUNK,), jnp.int32),
            pltpu.VMEM((CHUNK, EMBED_DIM), jnp.float32),
            pltpu.SemaphoreType.DMA,
        ],
        compiler_params=pltpu.CompilerParams(use_tc_tiling_on_sc=False),
    )(_emb_kernel)

    out_in, out_sup = run(in_flat, sup_flat, table)
    return (out_in.reshape(BATCH, SEQ, EMBED_DIM),
            out_sup.reshape(BATCH, SEQ, EMBED_DIM))


# speedup vs baseline: 1.3078x; 1.0140x over previous
import functools

import jax
import jax.numpy as jnp
from jax import lax
from jax.experimental import pallas as pl
from jax.experimental.pallas import tpu as pltpu
from jax.experimental.pallas import tpu_sc as plsc

VOCAB = 1000000
EMBED_DIM = 64
BATCH = 4096
SEQ = 50

_INFO = plsc.get_sparse_core_info()
NC, NS = _INFO.num_cores, _INFO.num_subcores
NW = NC * NS

N_ROWS = BATCH * SEQ
PER_W = N_ROWS // NW
CHUNK = 800
N_CHUNKS = PER_W // CHUNK
N_STEPS = 2 * N_CHUNKS
NBUF = 2


def _emb_kernel(in_idx_hbm, sup_idx_hbm, table_hbm, out_in_hbm, out_sup_hbm,
                idx_v, rows_v, gsems, wsems):
    wid = lax.axis_index("s") * NC + lax.axis_index("c")
    base_w = wid * PER_W

    pltpu.sync_copy(in_idx_hbm.at[pl.ds(wid * N_CHUNKS, N_CHUNKS)],
                    idx_v.at[pl.ds(0, N_CHUNKS)])
    pltpu.sync_copy(sup_idx_hbm.at[pl.ds(wid * N_CHUNKS, N_CHUNKS)],
                    idx_v.at[pl.ds(N_CHUNKS, N_CHUNKS)])

    writes = [None] * NBUF
    for step in range(N_STEPS):
        buf = step % NBUF
        tensor, c = divmod(step, N_CHUNKS)
        out_hbm = out_in_hbm if tensor == 0 else out_sup_hbm
        base = base_w + c * CHUNK
        if writes[buf] is not None:
            writes[buf].wait()
        pltpu.async_copy(table_hbm.at[idx_v.at[step]], rows_v.at[buf],
                         gsems.at[buf]).wait()
        writes[buf] = pltpu.async_copy(rows_v.at[buf],
                                       out_hbm.at[pl.ds(base, CHUNK)],
                                       wsems.at[buf])
    for w in writes:
        w.wait()


@jax.jit
def kernel(input, support, table):
    in_flat = input.reshape(N_ROWS // CHUNK, CHUNK).astype(jnp.int32)
    sup_flat = support.reshape(N_ROWS // CHUNK, CHUNK).astype(jnp.int32)

    run = functools.partial(
        pl.kernel,
        out_type=(
            jax.ShapeDtypeStruct((N_ROWS, EMBED_DIM), jnp.float32),
            jax.ShapeDtypeStruct((N_ROWS, EMBED_DIM), jnp.float32),
        ),
        mesh=plsc.VectorSubcoreMesh(core_axis_name="c", subcore_axis_name="s"),
        scratch_types=[
            pltpu.VMEM((N_STEPS, CHUNK), jnp.int32),
            pltpu.VMEM((NBUF, CHUNK, EMBED_DIM), jnp.float32),
            pltpu.SemaphoreType.DMA((NBUF,)),
            pltpu.SemaphoreType.DMA((NBUF,)),
        ],
        compiler_params=pltpu.CompilerParams(use_tc_tiling_on_sc=False),
    )(_emb_kernel)

    out_in, out_sup = run(in_flat, sup_flat, table)
    return (out_in.reshape(BATCH, SEQ, EMBED_DIM),
            out_sup.reshape(BATCH, SEQ, EMBED_DIM))
